# baseline (speedup 1.0000x reference)
import functools

import jax
import jax.numpy as jnp
import numpy as np
from jax import lax
from jax.experimental import pallas as pl
from jax.experimental.pallas import tpu as pltpu
from jax.experimental.pallas import tpu_sc as plsc

_ROWS = 16384
_COLS = 4096
_STRIDE = 64
_OFFSET = 7
_NOUT = 64
_TOTAL = _ROWS * _NOUT
_NC = 2
_NS = 16
_NW = _NC * _NS
_PER_W = _TOTAL // _NW

_mesh = plsc.VectorSubcoreMesh(core_axis_name="c", subcore_axis_name="s")


@functools.partial(
    pl.kernel,
    out_type=jax.ShapeDtypeStruct((_TOTAL,), jnp.float32),
    mesh=_mesh,
    scratch_types=[
        pltpu.VMEM((_PER_W,), jnp.int32),
        pltpu.VMEM((_PER_W,), jnp.float32),
        pltpu.SemaphoreType.DMA,
    ],
    compiler_params=pltpu.CompilerParams(use_tc_tiling_on_sc=False),
)
def _gather_col(x_hbm, idx_hbm, out_hbm, idx_v, vals_v, sem):
    wid = lax.axis_index("s") * _NC + lax.axis_index("c")
    base = wid * _PER_W
    pltpu.sync_copy(idx_hbm.at[pl.ds(base, _PER_W)], idx_v)
    pltpu.async_copy(x_hbm.at[idx_v], vals_v, sem).wait()
    pltpu.sync_copy(vals_v, out_hbm.at[pl.ds(base, _PER_W)])


_IDX_NP = np.arange(_TOTAL, dtype=np.int32) * _STRIDE + _OFFSET


def kernel(x):
    x_flat = x.reshape(_ROWS * _COLS)
    out = _gather_col(x_flat, jnp.asarray(_IDX_NP))
    return out.reshape(_ROWS, _NOUT)
